# baseline (speedup 1.0000x reference)
import jax
import jax.numpy as jnp
from jax.experimental import pallas as pl
from jax.experimental.pallas import tpu as pltpu

_TN = 1024


def _moe_kernel(x_ref, wr_ref, br_ref, we_ref, be_ref, out_ref, gates_ref):
    e = pl.program_id(1)
    n_e = pl.num_programs(1)
    E = gates_ref.shape[1]

    @pl.when(e == 0)
    def _compute_gates():
        logits = jnp.dot(x_ref[...], wr_ref[...],
                         preferred_element_type=jnp.float32)
        logits = logits + br_ref[...]
        m = jnp.max(logits, axis=1, keepdims=True)
        ex = jnp.exp(logits - m)
        gates_ref[...] = ex / jnp.sum(ex, axis=1, keepdims=True)

    y = jnp.dot(x_ref[...], we_ref[0], preferred_element_type=jnp.float32)
    g = gates_ref[...]
    eidx = jax.lax.broadcasted_iota(jnp.int32, (1, E), 1)
    ge = jnp.sum(jnp.where(eidx == e, g, 0.0), axis=1, keepdims=True)
    contrib = y * ge

    @pl.when(e == 0)
    def _init():
        out_ref[...] = contrib

    @pl.when(e > 0)
    def _acc():
        out_ref[...] += contrib

    @pl.when(e == n_e - 1)
    def _bias():
        out_ref[...] += jnp.dot(g, be_ref[...],
                                preferred_element_type=jnp.float32)


def kernel(x, Wr, br, We, be):
    N, D = x.shape
    E = We.shape[0]
    br2 = br.reshape(1, E)
    return pl.pallas_call(
        _moe_kernel,
        grid=(N // _TN, E),
        in_specs=[
            pl.BlockSpec((_TN, D), lambda i, e: (i, 0)),
            pl.BlockSpec((D, E), lambda i, e: (0, 0)),
            pl.BlockSpec((1, E), lambda i, e: (0, 0)),
            pl.BlockSpec((1, D, D), lambda i, e: (e, 0, 0)),
            pl.BlockSpec((E, D), lambda i, e: (0, 0)),
        ],
        out_specs=pl.BlockSpec((_TN, D), lambda i, e: (i, 0)),
        out_shape=jax.ShapeDtypeStruct((N, D), jnp.float32),
        scratch_shapes=[pltpu.VMEM((_TN, E), jnp.float32)],
    )(x, Wr, br2, We, be)
